# baseline (speedup 1.0000x reference)
import functools

import jax
import jax.numpy as jnp
from jax import lax
from jax.experimental import pallas as pl
from jax.experimental.pallas import tpu as pltpu
from jax.experimental.pallas import tpu_sc as plsc

NUM_SEGMENTS = 16
TOTAL_TOKENS = 32768
EMBED_DIM = 256
L = 16

NC = 2
NS = 16
TOK_PER_TILE = TOTAL_TOKENS // NS
DIM_PER_CORE = EMBED_DIM // NC
CHUNK = 128
NBUF = 4
NCHUNK = TOK_PER_TILE // CHUNK
NGRP = CHUNK // L
JGRP = DIM_PER_CORE // L

_mesh = plsc.VectorSubcoreMesh(core_axis_name="c", subcore_axis_name="s")


@functools.partial(
    pl.kernel,
    out_type=jax.ShapeDtypeStruct((NUM_SEGMENTS, EMBED_DIM), jnp.float32),
    mesh=_mesh,
    compiler_params=pltpu.CompilerParams(needs_layout_passes=False),
    scratch_types=[
        pltpu.VMEM((TOK_PER_TILE,), jnp.int32),
        pltpu.VMEM((TOK_PER_TILE,), jnp.float32),
        pltpu.VMEM((CHUNK, DIM_PER_CORE), jnp.float32),
        pltpu.VMEM((CHUNK, DIM_PER_CORE), jnp.float32),
        pltpu.VMEM((CHUNK, DIM_PER_CORE), jnp.float32),
        pltpu.VMEM((CHUNK, DIM_PER_CORE), jnp.float32),
        pltpu.VMEM(((NUM_SEGMENTS + 1) * DIM_PER_CORE,), jnp.float32),
        pltpu.VMEM(((NUM_SEGMENTS + 1) * L,), jnp.float32),
        pltpu.VMEM((1, DIM_PER_CORE), jnp.float32),
        pltpu.VMEM((NS, DIM_PER_CORE), jnp.float32),
        pltpu.VMEM((NS, NUM_SEGMENTS * L), jnp.float32),
        pltpu.VMEM((1, DIM_PER_CORE), jnp.float32),
        pltpu.VMEM_SHARED((NS, NUM_SEGMENTS * DIM_PER_CORE), jnp.float32),
        pltpu.VMEM_SHARED((NS, NUM_SEGMENTS * L), jnp.float32),
        pltpu.SemaphoreType.DMA,
        pltpu.SemaphoreType.DMA,
        pltpu.SemaphoreType.DMA,
        pltpu.SemaphoreType.DMA,
    ],
)
def _segment_embed_sc(emb, inten, seg, off, out,
                      seg_v, w_v, eb0, eb1, eb2, eb3, acc, accw, off_v,
                      cbuf, wbuf, outb, sh_all, sh_wall,
                      sem0, sem1, sem2, sem3):
    c = lax.axis_index("c")
    s = lax.axis_index("s")
    tok_base = s * TOK_PER_TILE
    dim_base = c * DIM_PER_CORE

    lane = jnp.arange(L, dtype=jnp.int32)
    zv = jnp.zeros((L,), jnp.float32)
    cols = [jnp.arange(L, dtype=jnp.int32) + j * L for j in range(JGRP)]
    ebufs = [eb0, eb1, eb2, eb3]
    sems = [sem0, sem1, sem2, sem3]

    def _src(g):
        return emb.at[pl.ds(tok_base + g * CHUNK, CHUNK),
                      pl.ds(dim_base, DIM_PER_CORE)]

    def _zero(i, _):
        for j in range(JGRP):
            acc[pl.ds(i * DIM_PER_CORE + j * L, L)] = zv
        accw[pl.ds(i * L, L)] = zv
        return _
    lax.fori_loop(0, NUM_SEGMENTS + 1, _zero, None)

    pltpu.sync_copy(seg.at[pl.ds(tok_base, TOK_PER_TILE)], seg_v)
    pltpu.sync_copy(inten.at[pl.ds(tok_base, TOK_PER_TILE)], w_v)
    pltpu.sync_copy(off.at[:, pl.ds(dim_base, DIM_PER_CORE)], off_v)

    for k in range(NBUF):
        pltpu.async_copy(_src(k), ebufs[k], sems[k])

    def _flush(cur, wacc, accs):
        plsc.addupdate_scatter(accw, [cur * L + lane], wacc)
        for j in range(JGRP):
            plsc.addupdate_scatter(acc, [cur * DIM_PER_CORE + cols[j]], accs[j])

    def _process(g, buf, carry):
        def _group(i, carry):
            cur = carry[0]
            gtok = g * CHUNK + i * L
            ltok = i * L
            sv = seg_v[pl.ds(gtok, L)]
            wv = w_v[pl.ds(gtok, L)]
            same = jnp.all(sv == cur)

            def _fast(cur, wacc, *accs):
                accs = list(accs)
                for t in range(L):
                    wsp = plsc.load_gather(
                        w_v, [jnp.full((L,), gtok + t, jnp.int32)])
                    for j in range(JGRP):
                        accs[j] = accs[j] + wsp * buf[ltok + t,
                                                      pl.ds(j * L, L)]
                return (cur, wacc + wv, *accs)

            def _slow(cur, wacc, *accs):
                _flush(cur, wacc, accs)
                plsc.addupdate_scatter(accw, [sv * L + lane], wv)
                for t in range(L):
                    tt = jnp.full((L,), gtok + t, jnp.int32)
                    wsp = plsc.load_gather(w_v, [tt])
                    ssp = plsc.load_gather(seg_v, [tt])
                    sbase = ssp * DIM_PER_CORE
                    for j in range(JGRP):
                        plsc.addupdate_scatter(
                            acc, [sbase + cols[j]],
                            wsp * buf[ltok + t, pl.ds(j * L, L)])
                ncur = plsc.load_gather(
                    seg_v, [jnp.full((L,), gtok + L - 1, jnp.int32)])
                return (ncur, zv) + tuple(zv for _ in range(JGRP))

            return lax.cond(same, _fast, _slow, *carry)

        return lax.fori_loop(0, NGRP, _group, carry)

    def _super(G, carry):
        for k in range(NBUF):
            g = NBUF * G + k
            pltpu.make_async_copy(_src(g), ebufs[k], sems[k]).wait()
            carry = _process(g, ebufs[k], carry)

            @pl.when(g + NBUF < NCHUNK)
            def _():
                pltpu.async_copy(_src(g + NBUF), ebufs[k], sems[k])
        return carry

    cur0 = jnp.full((L,), NUM_SEGMENTS, jnp.int32)
    carry = lax.fori_loop(0, NCHUNK // NBUF, _super,
                          (cur0, zv) + tuple(zv for _ in range(JGRP)))
    _flush(carry[0], carry[1], list(carry[2:]))

    pltpu.sync_copy(acc.at[pl.ds(0, NUM_SEGMENTS * DIM_PER_CORE)], sh_all.at[s])
    pltpu.sync_copy(accw.at[pl.ds(0, NUM_SEGMENTS * L)], sh_wall.at[s])
    plsc.subcore_barrier()

    pltpu.sync_copy(sh_all.at[:, pl.ds(s * DIM_PER_CORE, DIM_PER_CORE)], cbuf)
    pltpu.sync_copy(sh_wall, wbuf)

    wrow = wbuf[0, pl.ds(s * L, L)]
    for t in range(1, NS):
        wrow = wrow + wbuf[t, pl.ds(s * L, L)]
    den = jnp.full((L,), jnp.sum(wrow) + 1.0, jnp.float32)
    y = plsc.bitcast(jnp.int32(0x7EF311C3) - plsc.bitcast(den, jnp.int32),
                     jnp.float32)
    for _ in range(3):
        y = y * (2.0 - den * y)
    for j in range(JGRP):
        v = cbuf[0, pl.ds(j * L, L)]
        for t in range(1, NS):
            v = v + cbuf[t, pl.ds(j * L, L)]
        outb[0, pl.ds(j * L, L)] = (v + off_v[0, pl.ds(j * L, L)]) * y
    pltpu.sync_copy(outb, out.at[pl.ds(s, 1), pl.ds(dim_base, DIM_PER_CORE)])


def kernel(embeddings, intensities, segment_ids, offset_token):
    seg32 = segment_ids.astype(jnp.int32)
    return _segment_embed_sc(embeddings, intensities, seg32, offset_token)


# speedup vs baseline: 1.1627x; 1.1627x over previous
import functools

import jax
import jax.numpy as jnp
from jax import lax
from jax.experimental import pallas as pl
from jax.experimental.pallas import tpu as pltpu
from jax.experimental.pallas import tpu_sc as plsc

NUM_SEGMENTS = 16
TOTAL_TOKENS = 32768
EMBED_DIM = 256
L = 16

NC = 2
NS = 16
TOK_PER_TILE = TOTAL_TOKENS // NS
DIM_PER_CORE = EMBED_DIM // NC
CHUNK = 256
NBUF = 2
NCHUNK = TOK_PER_TILE // CHUNK
NGRP = CHUNK // L
JGRP = DIM_PER_CORE // L

_mesh = plsc.VectorSubcoreMesh(core_axis_name="c", subcore_axis_name="s")


@functools.partial(
    pl.kernel,
    out_type=jax.ShapeDtypeStruct((NUM_SEGMENTS, EMBED_DIM), jnp.float32),
    mesh=_mesh,
    compiler_params=pltpu.CompilerParams(needs_layout_passes=False),
    scratch_types=[
        pltpu.VMEM((TOK_PER_TILE,), jnp.int32),
        pltpu.VMEM((TOK_PER_TILE,), jnp.float32),
        pltpu.VMEM((CHUNK, DIM_PER_CORE), jnp.float32),
        pltpu.VMEM((CHUNK, DIM_PER_CORE), jnp.float32),
        pltpu.VMEM(((NUM_SEGMENTS + 1) * DIM_PER_CORE,), jnp.float32),
        pltpu.VMEM(((NUM_SEGMENTS + 1) * L,), jnp.float32),
        pltpu.VMEM((1, DIM_PER_CORE), jnp.float32),
        pltpu.VMEM((NS, DIM_PER_CORE), jnp.float32),
        pltpu.VMEM((NS, NUM_SEGMENTS * L), jnp.float32),
        pltpu.VMEM((1, DIM_PER_CORE), jnp.float32),
        pltpu.VMEM_SHARED((NS, NUM_SEGMENTS * DIM_PER_CORE), jnp.float32),
        pltpu.VMEM_SHARED((NS, NUM_SEGMENTS * L), jnp.float32),
        pltpu.SemaphoreType.DMA,
        pltpu.SemaphoreType.DMA,
    ],
)
def _segment_embed_sc(emb, inten, seg, off, out,
                      seg_v, w_v, eb0, eb1, acc, accw, off_v,
                      cbuf, wbuf, outb, sh_all, sh_wall,
                      sem0, sem1):
    c = lax.axis_index("c")
    s = lax.axis_index("s")
    tok_base = s * TOK_PER_TILE
    dim_base = c * DIM_PER_CORE

    lane = jnp.arange(L, dtype=jnp.int32)
    zv = jnp.zeros((L,), jnp.float32)
    cols = [jnp.arange(L, dtype=jnp.int32) + j * L for j in range(JGRP)]
    ebufs = [eb0, eb1]
    sems = [sem0, sem1]

    def _src(g):
        return emb.at[pl.ds(tok_base + g * CHUNK, CHUNK),
                      pl.ds(dim_base, DIM_PER_CORE)]

    def _zero(i, _):
        for j in range(JGRP):
            acc[pl.ds(i * DIM_PER_CORE + j * L, L)] = zv
        accw[pl.ds(i * L, L)] = zv
        return _
    lax.fori_loop(0, NUM_SEGMENTS + 1, _zero, None)

    pltpu.sync_copy(seg.at[pl.ds(tok_base, TOK_PER_TILE)], seg_v)
    pltpu.sync_copy(inten.at[pl.ds(tok_base, TOK_PER_TILE)], w_v)
    pltpu.sync_copy(off.at[:, pl.ds(dim_base, DIM_PER_CORE)], off_v)

    for k in range(NBUF):
        pltpu.async_copy(_src(k), ebufs[k], sems[k])

    def _flush(cur, wacc, accs):
        plsc.addupdate_scatter(accw, [cur * L + lane], wacc)
        for j in range(JGRP):
            plsc.addupdate_scatter(acc, [cur * DIM_PER_CORE + cols[j]], accs[j])

    def _process(g, buf, carry):
        def _group(i, carry):
            cur = carry[0]
            gtok = g * CHUNK + i * L
            ltok = i * L
            sv = seg_v[pl.ds(gtok, L)]
            wv = w_v[pl.ds(gtok, L)]
            same = jnp.all(sv == cur)

            def _fast(cur, wacc, *accs):
                accs = list(accs)
                for t in range(L):
                    wsp = plsc.load_gather(
                        w_v, [jnp.full((L,), gtok + t, jnp.int32)])
                    for j in range(JGRP):
                        accs[j] = accs[j] + wsp * buf[ltok + t,
                                                      pl.ds(j * L, L)]
                return (cur, wacc + wv, *accs)

            def _slow(cur, wacc, *accs):
                _flush(cur, wacc, accs)
                plsc.addupdate_scatter(accw, [sv * L + lane], wv)
                for t in range(L):
                    tt = jnp.full((L,), gtok + t, jnp.int32)
                    wsp = plsc.load_gather(w_v, [tt])
                    ssp = plsc.load_gather(seg_v, [tt])
                    sbase = ssp * DIM_PER_CORE
                    for j in range(JGRP):
                        plsc.addupdate_scatter(
                            acc, [sbase + cols[j]],
                            wsp * buf[ltok + t, pl.ds(j * L, L)])
                ncur = plsc.load_gather(
                    seg_v, [jnp.full((L,), gtok + L - 1, jnp.int32)])
                return (ncur, zv) + tuple(zv for _ in range(JGRP))

            return lax.cond(same, _fast, _slow, *carry)

        return lax.fori_loop(0, NGRP, _group, carry)

    def _super(G, carry):
        for k in range(NBUF):
            g = NBUF * G + k
            pltpu.make_async_copy(_src(g), ebufs[k], sems[k]).wait()
            carry = _process(g, ebufs[k], carry)

            @pl.when(g + NBUF < NCHUNK)
            def _():
                pltpu.async_copy(_src(g + NBUF), ebufs[k], sems[k])
        return carry

    cur0 = jnp.full((L,), NUM_SEGMENTS, jnp.int32)
    carry = lax.fori_loop(0, NCHUNK // NBUF, _super,
                          (cur0, zv) + tuple(zv for _ in range(JGRP)))
    _flush(carry[0], carry[1], list(carry[2:]))

    pltpu.sync_copy(acc.at[pl.ds(0, NUM_SEGMENTS * DIM_PER_CORE)], sh_all.at[s])
    pltpu.sync_copy(accw.at[pl.ds(0, NUM_SEGMENTS * L)], sh_wall.at[s])
    plsc.subcore_barrier()

    pltpu.sync_copy(sh_all.at[:, pl.ds(s * DIM_PER_CORE, DIM_PER_CORE)], cbuf)
    pltpu.sync_copy(sh_wall, wbuf)

    wrow = wbuf[0, pl.ds(s * L, L)]
    for t in range(1, NS):
        wrow = wrow + wbuf[t, pl.ds(s * L, L)]
    den = jnp.full((L,), jnp.sum(wrow) + 1.0, jnp.float32)
    y = plsc.bitcast(jnp.int32(0x7EF311C3) - plsc.bitcast(den, jnp.int32),
                     jnp.float32)
    for _ in range(3):
        y = y * (2.0 - den * y)
    for j in range(JGRP):
        v = cbuf[0, pl.ds(j * L, L)]
        for t in range(1, NS):
            v = v + cbuf[t, pl.ds(j * L, L)]
        outb[0, pl.ds(j * L, L)] = (v + off_v[0, pl.ds(j * L, L)]) * y
    pltpu.sync_copy(outb, out.at[pl.ds(s, 1), pl.ds(dim_base, DIM_PER_CORE)])


def kernel(embeddings, intensities, segment_ids, offset_token):
    seg32 = segment_ids.astype(jnp.int32)
    return _segment_embed_sc(embeddings, intensities, seg32, offset_token)
